# baseline (speedup 1.0000x reference)
import functools

import jax
import jax.numpy as jnp
from jax.experimental import pallas as pl

B, C, T = 128, 64, 1024
EMB = 32
TK = 64
POOL = 4
TOPK = 8
HID = 64
HEADS = 4
G1 = HID * HEADS
G2 = HID
NCLS = 2
TP = T // POOL
E = C * TOPK
BB = 8
NR = BB * C
NE = BB * E

_dot = functools.partial(jnp.dot, precision=jax.lax.Precision.HIGHEST,
                         preferred_element_type=jnp.float32)


def _bf(a):
    return a.astype(jnp.bfloat16)


def _dot_bf(a, b):
    return jnp.dot(_bf(a), _bf(b), preferred_element_type=jnp.float32)


def _dot_t(a, b):
    return jax.lax.dot_general(
        a, b, (((0,), (0,)), ((), ())),
        precision=jax.lax.Precision.HIGHEST,
        preferred_element_type=jnp.float32)


def _elu(v):
    return jnp.where(v > 0, v, jnp.exp(v) - 1.0)


def _gat_layer(h, oh2, ew_e, wl, bl, wr, br, we, att_mat, e_mat, bias,
               heads, g):
    xl = _dot_bf(h, wl) + bl
    xr = _dot_bf(h, wr) + br
    xr_g = jnp.concatenate(
        [_dot(oh2[b * E:(b + 1) * E], xr[b * C:(b + 1) * C])
         for b in range(BB)], axis=0)
    xl_rep = jnp.broadcast_to(xl[:, None, :], (NR, TOPK, g)).reshape(NE, g)
    m = (xl_rep + xr_g
         + _bf(ew_e).astype(jnp.float32) * _bf(we).astype(jnp.float32))
    m = jnp.where(m > 0, m, 0.2 * m)
    logit = _dot(m, att_mat)
    gmax = jnp.max(logit, axis=0, keepdims=True)
    aexp = jnp.exp(logit - gmax)
    den_e = jnp.concatenate(
        [_dot(oh2[b * E:(b + 1) * E],
              _dot_t(oh2[b * E:(b + 1) * E], aexp[b * E:(b + 1) * E]))
         for b in range(BB)], axis=0)
    wgt = aexp / (den_e + 1e-16)
    if heads > 1:
        wx = _dot(wgt, e_mat)
    else:
        wx = wgt
    wm = wx * xl_rep
    out = jnp.concatenate(
        [_dot_t(oh2[b * E:(b + 1) * E], wm[b * E:(b + 1) * E])
         for b in range(BB)], axis=0)
    return out + bias


def _main_body(x_ref, wconv_ref, s1_ref, o1_ref, pmat_ref,
               amul_ref, dadd_ref,
               wl1_ref, bl1_ref, wr1_ref, br1_ref, we1_ref, attm1_ref,
               em1_ref, bias1_ref,
               wl2_ref, bl2_ref, wr2_ref, br2_ref, we2_ref, attm2_ref,
               bias2_ref, out_ref):
    xt = jnp.transpose(x_ref[...].reshape(NR, T))
    zero32 = jnp.zeros((TK // 2, NR), jnp.float32)
    xp = _bf(jnp.concatenate([zero32, xt, zero32],
                             axis=0)).astype(jnp.float32)
    w = _bf(wconv_ref[...]).astype(jnp.float32)
    RCH = 64
    tiles = []
    for t0 in range(0, T, RCH):
        acc = jnp.zeros((RCH, NR), jnp.float32)
        for k in range(TK):
            acc = acc + xp[t0 + k:t0 + k + RCH, :] * w[k:k + 1, :]
        tiles.append(acc)
    y = jnp.concatenate(tiles, axis=0)
    y = _elu(y * s1_ref[...] + o1_ref[...])
    pooled = _dot(pmat_ref[...], y)

    nf_rows = []
    for e in range(EMB):
        ze = _elu(pooled * amul_ref[e, 0] + dadd_ref[e, 0])
        nf_rows.append(jnp.mean(ze, axis=0, keepdims=True))
    nft = jnp.concatenate(nf_rows, axis=0)
    nf = jnp.concatenate(
        [jnp.transpose(nft[:, b * C:(b + 1) * C]) for b in range(BB)],
        axis=0)

    mu = jnp.mean(xt, axis=0, keepdims=True)
    xm = xt - mu
    ss = jnp.sum(xm * xm, axis=0, keepdims=True)
    std = jnp.sqrt(ss / (T - 1))
    xs = xm / (std + 1e-8)
    xsb = _bf(xs)
    corr = jnp.concatenate(
        [jax.lax.dot_general(
            xsb[:, b * C:(b + 1) * C], xsb[:, b * C:(b + 1) * C],
            (((0,), (0,)), ((), ())),
            preferred_element_type=jnp.float32)
         for b in range(BB)], axis=0) * (1.0 / (T - 1))
    corr = jnp.clip(corr, -1.0, 1.0)
    iota_c = jax.lax.broadcasted_iota(jnp.int32, (NR, C), 1)
    iota_r = jax.lax.broadcasted_iota(
        jnp.int32, (BB, C, C), 1).reshape(NR, C)
    score = jnp.where(iota_c == iota_r, -1e30, jnp.abs(corr))
    idx_cols = []
    sc = score
    for _ in range(TOPK):
        mx = jnp.max(sc, axis=1, keepdims=True)
        cand = jnp.where(sc >= mx, iota_c, C)
        ij = jnp.min(cand, axis=1, keepdims=True)
        oh = iota_c == ij
        idx_cols.append(ij)
        sc = jnp.where(oh, -1e30, sc)
    idxs = jnp.concatenate(idx_cols, axis=1)

    i3 = jax.lax.broadcasted_iota(jnp.int32, (NR, TOPK, C), 2)
    oh2 = (i3 == idxs[:, :, None]).astype(jnp.float32).reshape(NE, C)
    corr_rep = jnp.broadcast_to(corr[:, None, :], (NR, TOPK, C)).reshape(NE, C)
    ew_e = jnp.sum(oh2 * corr_rep, axis=1, keepdims=True)

    h1 = _elu(_gat_layer(nf, oh2, ew_e, wl1_ref[...], bl1_ref[...],
                         wr1_ref[...], br1_ref[...], we1_ref[...],
                         attm1_ref[...], em1_ref[...], bias1_ref[...],
                         HEADS, G1))
    h2 = _elu(_gat_layer(h1, oh2, ew_e, wl2_ref[...], bl2_ref[...],
                         wr2_ref[...], br2_ref[...], we2_ref[...],
                         attm2_ref[...], None, bias2_ref[...],
                         1, G2))

    out_ref[0] = jnp.mean(h2.reshape(BB, C, G2), axis=1)


def _head_body(g_ref, fc1w_ref, fc1b_ref, fc2w_ref, fc2b_ref,
               p1w_ref, p1b_ref, p2w_ref, p2b_ref, a1_ref, a2_ref,
               c1w_ref, c1b_ref, c2w_ref, c2b_ref, out_ref):
    g = g_ref[...]
    a1 = jnp.tanh(_dot_bf(g, fc1w_ref[...]) + fc1b_ref[...])
    a2 = jnp.tanh(_dot_bf(g, fc2w_ref[...]) + fc2b_ref[...])
    g = (g + (_dot_bf(a1, p1w_ref[...]) + p1b_ref[...]) * a1_ref[...]
         + (_dot_bf(a2 * a2, p2w_ref[...]) + p2b_ref[...]) * a2_ref[...])
    z = jnp.maximum(_dot_bf(g, c1w_ref[...]) + c1b_ref[...], 0.0)
    out_ref[...] = _dot_bf(z, c2w_ref[...]) + c2b_ref[...]


def kernel(x, params):
    p = params
    f32 = jnp.float32

    s1v = p['bn1_g'] / jnp.sqrt(p['bn1_v'] + 1e-5)
    o1v = p['bn1_b'] - p['bn1_m'] * s1v
    s1 = jnp.tile(s1v, BB).reshape(1, NR)
    o1 = jnp.tile(o1v, BB).reshape(1, NR)
    wconv = jnp.tile(p['dw_w'].reshape(C, TK).T, (1, BB))
    s2 = p['bn2_g'] / jnp.sqrt(p['bn2_v'] + 1e-5)
    amul = (p['pw_w'] * s2).reshape(EMB, 1)
    dadd = (p['bn2_b'] - p['bn2_m'] * s2).reshape(EMB, 1)
    pmat = ((jnp.arange(T)[None, :] // POOL == jnp.arange(TP)[:, None])
            .astype(f32) * (1.0 / POOL))
    attm1 = (p['g1_att'][:, :, None]
             * jnp.eye(HEADS, dtype=f32)[:, None, :]).reshape(G1, HEADS)
    em1 = (jnp.eye(HEADS, dtype=f32)[:, :, None]
           * jnp.ones((1, 1, HID), f32)).reshape(HEADS, G1)
    attm2 = p['g2_att'].reshape(G2, 1)

    row = lambda v: v.reshape(1, -1)
    main_inputs = [
        x, wconv, s1, o1, pmat, amul, dadd,
        p['g1_wl'], row(p['g1_bl']), p['g1_wr'], row(p['g1_br']),
        row(p['g1_we']), attm1, em1, row(p['g1_bias']),
        p['g2_wl'], row(p['g2_bl']), p['g2_wr'], row(p['g2_br']),
        row(p['g2_we']), attm2, row(p['g2_bias']),
    ]
    const = lambda a: pl.BlockSpec(a.shape, lambda b: (0,) * a.ndim)
    in_specs = [pl.BlockSpec((BB, C, T), lambda b: (b, 0, 0))]
    in_specs += [const(a) for a in main_inputs[1:]]
    gpool = pl.pallas_call(
        _main_body,
        grid=(B // BB,),
        in_specs=in_specs,
        out_specs=pl.BlockSpec((1, BB, G2), lambda b: (b, 0, 0)),
        out_shape=jax.ShapeDtypeStruct((B // BB, BB, G2), f32),
    )(*main_inputs)
    gpool = gpool.reshape(B, G2)

    npad = 128
    c2w = jnp.zeros((G2, npad), f32).at[:, :NCLS].set(p['c2_w'])
    c2b = jnp.zeros((1, npad), f32).at[:, :NCLS].set(p['c2_b'])
    head_inputs = [
        gpool, p['fc1_w'], row(p['fc1_b']), p['fc2_w'], row(p['fc2_b']),
        p['p1_w'], row(p['p1_b']), p['p2_w'], row(p['p2_b']),
        row(p['alpha1']), row(p['alpha2']),
        p['c1_w'], row(p['c1_b']), c2w, c2b,
    ]
    out = pl.pallas_call(
        _head_body,
        out_shape=jax.ShapeDtypeStruct((B, npad), f32),
    )(*head_inputs)
    return out[:, :NCLS]


# speedup vs baseline: 1.1256x; 1.1256x over previous
import functools

import jax
import jax.numpy as jnp
from jax.experimental import pallas as pl

B, C, T = 128, 64, 1024
EMB = 32
TK = 64
POOL = 4
TOPK = 8
HID = 64
HEADS = 4
G1 = HID * HEADS
G2 = HID
NCLS = 2
TP = T // POOL
E = C * TOPK
BB = 4
NR = BB * C
NE = BB * E

_dot = functools.partial(jnp.dot, precision=jax.lax.Precision.HIGHEST,
                         preferred_element_type=jnp.float32)


def _bf(a):
    return a.astype(jnp.bfloat16)


def _dot_bf(a, b):
    return jnp.dot(_bf(a), _bf(b), preferred_element_type=jnp.float32)


def _dot_t(a, b):
    return jax.lax.dot_general(
        a, b, (((0,), (0,)), ((), ())),
        precision=jax.lax.Precision.HIGHEST,
        preferred_element_type=jnp.float32)


def _elu(v):
    return jnp.where(v > 0, v, jnp.exp(v) - 1.0)


def _gat_layer(h, oh2, ew_e, wl, bl, wr, br, we, att_mat, e_mat, bias,
               heads, g):
    xl = _dot_bf(h, wl) + bl
    xr = _dot_bf(h, wr) + br
    xr_g = jnp.concatenate(
        [_dot(oh2[b * E:(b + 1) * E], xr[b * C:(b + 1) * C])
         for b in range(BB)], axis=0)
    xl_rep = jnp.broadcast_to(xl[:, None, :], (NR, TOPK, g)).reshape(NE, g)
    m = (xl_rep + xr_g
         + _bf(ew_e).astype(jnp.float32) * _bf(we).astype(jnp.float32))
    m = jnp.where(m > 0, m, 0.2 * m)
    logit = _dot(m, att_mat)
    gmax = jnp.max(logit, axis=0, keepdims=True)
    aexp = jnp.exp(logit - gmax)
    den_e = jnp.concatenate(
        [_dot(oh2[b * E:(b + 1) * E],
              _dot_t(oh2[b * E:(b + 1) * E], aexp[b * E:(b + 1) * E]))
         for b in range(BB)], axis=0)
    wgt = aexp / (den_e + 1e-16)
    if heads > 1:
        wx = _dot(wgt, e_mat)
    else:
        wx = wgt
    wm = wx * xl_rep
    out = jnp.concatenate(
        [_dot_t(oh2[b * E:(b + 1) * E], wm[b * E:(b + 1) * E])
         for b in range(BB)], axis=0)
    return out + bias


def _main_body(x_ref, wconv_ref, s1_ref, o1_ref, pmat_ref,
               amul_ref, dadd_ref,
               wl1_ref, bl1_ref, wr1_ref, br1_ref, we1_ref, attm1_ref,
               em1_ref, bias1_ref,
               wl2_ref, bl2_ref, wr2_ref, br2_ref, we2_ref, attm2_ref,
               bias2_ref, out_ref):
    xt = jnp.transpose(x_ref[...].reshape(NR, T))
    zero32 = jnp.zeros((TK // 2, NR), jnp.float32)
    xp = _bf(jnp.concatenate([zero32, xt, zero32],
                             axis=0)).astype(jnp.float32)
    w = _bf(wconv_ref[...]).astype(jnp.float32)
    RCH = 64
    tiles = []
    for t0 in range(0, T, RCH):
        acc = jnp.zeros((RCH, NR), jnp.float32)
        for k in range(TK):
            acc = acc + xp[t0 + k:t0 + k + RCH, :] * w[k:k + 1, :]
        tiles.append(acc)
    y = jnp.concatenate(tiles, axis=0)
    y = _elu(y * s1_ref[...] + o1_ref[...])
    pooled = _dot(pmat_ref[...], y)

    nf_rows = []
    for e in range(EMB):
        ze = _elu(pooled * amul_ref[e, 0] + dadd_ref[e, 0])
        nf_rows.append(jnp.mean(ze, axis=0, keepdims=True))
    nft = jnp.concatenate(nf_rows, axis=0)
    nf = jnp.concatenate(
        [jnp.transpose(nft[:, b * C:(b + 1) * C]) for b in range(BB)],
        axis=0)

    mu = jnp.mean(xt, axis=0, keepdims=True)
    xm = xt - mu
    ss = jnp.sum(xm * xm, axis=0, keepdims=True)
    std = jnp.sqrt(ss / (T - 1))
    xs = xm / (std + 1e-8)
    xsb = _bf(xs)
    corr = jnp.concatenate(
        [jax.lax.dot_general(
            xsb[:, b * C:(b + 1) * C], xsb[:, b * C:(b + 1) * C],
            (((0,), (0,)), ((), ())),
            preferred_element_type=jnp.float32)
         for b in range(BB)], axis=0) * (1.0 / (T - 1))
    corr = jnp.clip(corr, -1.0, 1.0)
    iota_c = jax.lax.broadcasted_iota(jnp.int32, (NR, C), 1)
    iota_r = jax.lax.broadcasted_iota(
        jnp.int32, (BB, C, C), 1).reshape(NR, C)
    score = jnp.where(iota_c == iota_r, -1e30, jnp.abs(corr))
    idx_cols = []
    sc = score
    for _ in range(TOPK):
        mx = jnp.max(sc, axis=1, keepdims=True)
        cand = jnp.where(sc >= mx, iota_c, C)
        ij = jnp.min(cand, axis=1, keepdims=True)
        oh = iota_c == ij
        idx_cols.append(ij)
        sc = jnp.where(oh, -1e30, sc)
    idxs = jnp.concatenate(idx_cols, axis=1)

    i3 = jax.lax.broadcasted_iota(jnp.int32, (NR, TOPK, C), 2)
    oh2 = (i3 == idxs[:, :, None]).astype(jnp.float32).reshape(NE, C)
    corr_rep = jnp.broadcast_to(corr[:, None, :], (NR, TOPK, C)).reshape(NE, C)
    ew_e = jnp.sum(oh2 * corr_rep, axis=1, keepdims=True)

    h1 = _elu(_gat_layer(nf, oh2, ew_e, wl1_ref[...], bl1_ref[...],
                         wr1_ref[...], br1_ref[...], we1_ref[...],
                         attm1_ref[...], em1_ref[...], bias1_ref[...],
                         HEADS, G1))
    h2 = _elu(_gat_layer(h1, oh2, ew_e, wl2_ref[...], bl2_ref[...],
                         wr2_ref[...], br2_ref[...], we2_ref[...],
                         attm2_ref[...], None, bias2_ref[...],
                         1, G2))

    out_ref[0] = jnp.mean(h2.reshape(BB, C, G2), axis=1)


def _head_body(g_ref, fc1w_ref, fc1b_ref, fc2w_ref, fc2b_ref,
               p1w_ref, p1b_ref, p2w_ref, p2b_ref, a1_ref, a2_ref,
               c1w_ref, c1b_ref, c2w_ref, c2b_ref, out_ref):
    g = g_ref[...]
    a1 = jnp.tanh(_dot_bf(g, fc1w_ref[...]) + fc1b_ref[...])
    a2 = jnp.tanh(_dot_bf(g, fc2w_ref[...]) + fc2b_ref[...])
    g = (g + (_dot_bf(a1, p1w_ref[...]) + p1b_ref[...]) * a1_ref[...]
         + (_dot_bf(a2 * a2, p2w_ref[...]) + p2b_ref[...]) * a2_ref[...])
    z = jnp.maximum(_dot_bf(g, c1w_ref[...]) + c1b_ref[...], 0.0)
    out_ref[...] = _dot_bf(z, c2w_ref[...]) + c2b_ref[...]


def kernel(x, params):
    p = params
    f32 = jnp.float32

    s1v = p['bn1_g'] / jnp.sqrt(p['bn1_v'] + 1e-5)
    o1v = p['bn1_b'] - p['bn1_m'] * s1v
    s1 = jnp.tile(s1v, BB).reshape(1, NR)
    o1 = jnp.tile(o1v, BB).reshape(1, NR)
    wconv = jnp.tile(p['dw_w'].reshape(C, TK).T, (1, BB))
    s2 = p['bn2_g'] / jnp.sqrt(p['bn2_v'] + 1e-5)
    amul = (p['pw_w'] * s2).reshape(EMB, 1)
    dadd = (p['bn2_b'] - p['bn2_m'] * s2).reshape(EMB, 1)
    pmat = ((jnp.arange(T)[None, :] // POOL == jnp.arange(TP)[:, None])
            .astype(f32) * (1.0 / POOL))
    attm1 = (p['g1_att'][:, :, None]
             * jnp.eye(HEADS, dtype=f32)[:, None, :]).reshape(G1, HEADS)
    em1 = (jnp.eye(HEADS, dtype=f32)[:, :, None]
           * jnp.ones((1, 1, HID), f32)).reshape(HEADS, G1)
    attm2 = p['g2_att'].reshape(G2, 1)

    row = lambda v: v.reshape(1, -1)
    main_inputs = [
        x, wconv, s1, o1, pmat, amul, dadd,
        p['g1_wl'], row(p['g1_bl']), p['g1_wr'], row(p['g1_br']),
        row(p['g1_we']), attm1, em1, row(p['g1_bias']),
        p['g2_wl'], row(p['g2_bl']), p['g2_wr'], row(p['g2_br']),
        row(p['g2_we']), attm2, row(p['g2_bias']),
    ]
    const = lambda a: pl.BlockSpec(a.shape, lambda b: (0,) * a.ndim)
    in_specs = [pl.BlockSpec((BB, C, T), lambda b: (b, 0, 0))]
    in_specs += [const(a) for a in main_inputs[1:]]
    gpool = pl.pallas_call(
        _main_body,
        grid=(B // BB,),
        in_specs=in_specs,
        out_specs=pl.BlockSpec((1, BB, G2), lambda b: (b, 0, 0)),
        out_shape=jax.ShapeDtypeStruct((B // BB, BB, G2), f32),
    )(*main_inputs)
    gpool = gpool.reshape(B, G2)

    npad = 128
    c2w = jnp.zeros((G2, npad), f32).at[:, :NCLS].set(p['c2_w'])
    c2b = jnp.zeros((1, npad), f32).at[:, :NCLS].set(p['c2_b'])
    head_inputs = [
        gpool, p['fc1_w'], row(p['fc1_b']), p['fc2_w'], row(p['fc2_b']),
        p['p1_w'], row(p['p1_b']), p['p2_w'], row(p['p2_b']),
        row(p['alpha1']), row(p['alpha2']),
        p['c1_w'], row(p['c1_b']), c2w, c2b,
    ]
    out = pl.pallas_call(
        _head_body,
        out_shape=jax.ShapeDtypeStruct((B, npad), f32),
    )(*head_inputs)
    return out[:, :NCLS]
